# baseline (speedup 1.0000x reference)
import functools

import jax
import jax.numpy as jnp
from jax import lax
from jax.experimental import pallas as pl
from jax.experimental.pallas import tpu as pltpu
from jax.experimental.pallas import tpu_sc as plsc

BATCH = 1024
SEQ = 200
HID = 64
LANES = 16
NC = 2
NS = 16
NW = NC * NS
TOK = BATCH * SEQ
TPW = TOK // NW
CHUNK = 2 * SEQ
NCHUNK = TPW // CHUNK
GROUPS = CHUNK // LANES


def _body(x_hbm, tab_hbm, pos_hbm, out_hbm, idx_v, rows_v, pos_v, gsem):
    wid = lax.axis_index("s") * NC + lax.axis_index("c")
    base_w = wid * TPW

    pltpu.sync_copy(pos_hbm, pos_v.at[pl.ds(0, SEQ)])
    pltpu.sync_copy(pos_hbm, pos_v.at[pl.ds(SEQ, SEQ)])

    pltpu.sync_copy(x_hbm.at[pl.ds(wid * (TPW // LANES), TPW // LANES)], idx_v)

    for c in range(NCHUNK):
        base = base_w + c * CHUNK

        cps = [
            pltpu.async_copy(
                tab_hbm.at[idx_v.at[c * GROUPS + k]],
                rows_v.at[pl.ds(k * LANES, LANES)],
                gsem,
            )
            for k in range(GROUPS)
        ]
        for cp in cps:
            cp.wait()

        def mask_body(g, _):
            v = idx_v[c * GROUPS + g, :]
            m = v == 1
            cnt = jnp.sum(m.astype(jnp.int32))

            @pl.when(cnt > 0)
            def _():
                ridx = g * LANES + lax.iota(jnp.int32, LANES)
                zeros = jnp.zeros((LANES,), jnp.float32)

                def zb(d, _):
                    plsc.store_scatter(
                        rows_v,
                        [ridx, jnp.full((LANES,), d, jnp.int32)],
                        zeros,
                        mask=m,
                    )
                    return 0

                lax.fori_loop(0, HID, zb, 0)

            return 0

        lax.fori_loop(0, GROUPS, mask_body, 0)

        def add_body(t, _):
            for d in range(HID // LANES):
                sl = pl.ds(d * LANES, LANES)
                plsc.addupdate(rows_v.at[t, sl], pos_v[t, sl])
            return 0

        lax.fori_loop(0, CHUNK, add_body, 0)

        pltpu.sync_copy(rows_v, out_hbm.at[pl.ds(base, CHUNK)])


@jax.jit
def _run(xf, table, pos_emb):
    mesh = plsc.VectorSubcoreMesh(core_axis_name="c", subcore_axis_name="s")
    f = pl.kernel(
        _body,
        mesh=mesh,
        compiler_params=pltpu.CompilerParams(
            use_tc_tiling_on_sc=False, needs_layout_passes=False
        ),
        out_type=jax.ShapeDtypeStruct((TOK, HID), jnp.float32),
        scratch_types=[
            pltpu.VMEM((TPW // LANES, LANES), jnp.int32),
            pltpu.VMEM((CHUNK, HID), jnp.float32),
            pltpu.VMEM((CHUNK, HID), jnp.float32),
            pltpu.SemaphoreType.DMA,
        ],
    )
    return f(xf, table, pos_emb)


def kernel(x, table, pos_emb):
    xf = x.astype(jnp.int32).reshape(TOK // LANES, LANES)
    out = _run(xf, table, pos_emb)
    return out.reshape(BATCH, SEQ, HID)
